# Initial kernel scaffold; baseline (speedup 1.0000x reference)
import jax
import jax.numpy as jnp
from jax.experimental import pallas as pl


def kernel(x, edge_idx, spec_domain, weight, bias, mu, sig):
    raise NotImplementedError("write your pallas kernel here")


# speedup vs baseline: 2.4451x; 2.4451x over previous
import functools

import jax
import jax.numpy as jnp
from jax import lax
from jax.experimental import pallas as pl
from jax.experimental.pallas import tpu as pltpu
from jax.experimental.pallas import tpu_sc as plsc

N = 10000
E = 320000
DF = 128
KER = 4
EMB = 16

NC = 2
NS = 16
NW = NC * NS
EPT = E // NW
B = 200
NBATCH = EPT // B
SUB = 40
NSUB = B // SUB
N_PAD = 10240
ROWS_PER_TILE = N_PAD // NS
ZROWS = 64


def _gather_spec_body(srcp, dstp, spec, d0_out, d1_out,
                      sidx, didx, d0, d1, sem):
    cid = lax.axis_index("c")
    sid = lax.axis_index("s")
    wid = cid * NS + sid

    def batch_body(i, carry):
        base = wid * EPT + i * B
        for j in range(NSUB):
            pltpu.sync_copy(srcp.at[pl.ds(base + j * SUB, SUB)], sidx.at[j])
            pltpu.sync_copy(dstp.at[pl.ds(base + j * SUB, SUB)], didx.at[j])
        descs = []
        for j in range(NSUB):
            descs.append(pltpu.async_copy(
                spec.at[sidx.at[j]], d0.at[pl.ds(j * SUB, SUB)], sem))
            descs.append(pltpu.async_copy(
                spec.at[didx.at[j]], d1.at[pl.ds(j * SUB, SUB)], sem))
        for d in descs:
            d.wait()
        pltpu.sync_copy(d0, d0_out.at[pl.ds(base, B)])
        pltpu.sync_copy(d1, d1_out.at[pl.ds(base, B)])
        return carry
    lax.fori_loop(0, NBATCH, batch_body, 0)


VB = 2560


def _vals_body(d0_ref, d1_ref, mu_ref, sig_ref, o_ref):
    diff = d0_ref[...] - d1_ref[...]
    sq = jnp.sum(diff * diff, axis=1)
    p = jnp.dot(diff, mu_ref[...], preferred_element_type=jnp.float32)
    for k in range(KER):
        m2k = jnp.sum(mu_ref[:, k] * mu_ref[:, k])
        qq = p[:, k] - 0.5 * (sq + m2k)
        o_ref[k, :] = jnp.exp(sig_ref[0, k] * qq)


def _agg_body(srcp, dstp, x, vals, t_out,
              sidx, didx, vb, xr, zbuf, acc, sem):
    cid = lax.axis_index("c")
    sid = lax.axis_index("s")
    wid = cid * NS + sid

    def zb_body(i, c):
        for r in range(DF // 16):
            zbuf[i, pl.ds(16 * r, 16)] = jnp.zeros((16,), jnp.float32)
        return c
    lax.fori_loop(0, ZROWS, zb_body, 0)

    for k in range(KER):
        for jj in range(ROWS_PER_TILE // ZROWS):
            pltpu.sync_copy(zbuf, acc.at[pl.ds(sid * ROWS_PER_TILE + jj * ZROWS, ZROWS)])
        plsc.subcore_barrier()

        def batch_body(i, carry):
            base = wid * EPT + i * B
            for j in range(NSUB):
                pltpu.sync_copy(srcp.at[pl.ds(base + j * SUB, SUB)], sidx.at[j])
                pltpu.sync_copy(dstp.at[pl.ds(base + j * SUB, SUB)], didx.at[j])
            pltpu.sync_copy(vals.at[pl.ds(k * E + base, B)], vb)
            descs = [pltpu.async_copy(x.at[didx.at[j]],
                                      xr.at[pl.ds(j * SUB, SUB)], sem)
                     for j in range(NSUB)]
            for d in descs:
                d.wait()

            def scale_body(g, c):
                vvec = vb[pl.ds(g * 16, 16)]
                for t in range(16):
                    ve = vvec[t]
                    e = g * 16 + t
                    for r in range(DF // 16):
                        xr[e, pl.ds(16 * r, 16)] = xr[e, pl.ds(16 * r, 16)] * ve
                return c
            lax.fori_loop(0, B // 16, scale_body, 0)

            for j in range(NSUB):
                pltpu.sync_copy(xr.at[pl.ds(j * SUB, SUB)],
                                acc.at[sidx.at[j]], add=True)
            return carry
        lax.fori_loop(0, NBATCH, batch_body, 0)
        plsc.subcore_barrier()
        pltpu.sync_copy(acc.at[pl.ds(sid * ROWS_PER_TILE, ROWS_PER_TILE)],
                        t_out.at[k, cid, pl.ds(sid * ROWS_PER_TILE, ROWS_PER_TILE)])
        plsc.subcore_barrier()


BN = 400


def _mm_body(t_ref, w_ref, b_ref, o_ref):
    acc = jnp.zeros((BN, DF), jnp.float32)
    for k in range(KER):
        acc = acc + jnp.dot(t_ref[k, 0] + t_ref[k, 1], w_ref[k],
                            preferred_element_type=jnp.float32)
    o_ref[...] = acc + b_ref[...]


def kernel(x, edge_idx, spec_domain, weight, bias, mu, sig):
    src = edge_idx[0]
    dst = edge_idx[1]
    wt = jnp.transpose(weight, (2, 0, 1))
    b2 = bias[None, :]

    mesh = plsc.VectorSubcoreMesh(core_axis_name="c", subcore_axis_name="s")

    gather_spec = pl.kernel(
        _gather_spec_body,
        out_type=(jax.ShapeDtypeStruct((E, EMB), jnp.float32),
                  jax.ShapeDtypeStruct((E, EMB), jnp.float32)),
        mesh=mesh,
        scratch_types=[
            pltpu.VMEM((NSUB, SUB), jnp.int32),
            pltpu.VMEM((NSUB, SUB), jnp.int32),
            pltpu.VMEM((B, EMB), jnp.float32),
            pltpu.VMEM((B, EMB), jnp.float32),
            pltpu.SemaphoreType.DMA,
        ],
        compiler_params=pltpu.CompilerParams(use_tc_tiling_on_sc=False),
    )
    d0f, d1f = gather_spec(src, dst, spec_domain)

    vals4 = pl.pallas_call(
        _vals_body,
        out_shape=jax.ShapeDtypeStruct((KER, E), jnp.float32),
        grid=(E // VB,),
        in_specs=[
            pl.BlockSpec((VB, EMB), lambda i: (i, 0)),
            pl.BlockSpec((VB, EMB), lambda i: (i, 0)),
            pl.BlockSpec((EMB, KER), lambda i: (0, 0)),
            pl.BlockSpec((1, KER), lambda i: (0, 0)),
        ],
        out_specs=pl.BlockSpec((KER, VB), lambda i: (0, i)),
    )(d0f, d1f, mu, sig)
    vals = vals4.reshape(-1)

    agg = pl.kernel(
        _agg_body,
        out_type=jax.ShapeDtypeStruct((KER, NC, N_PAD, DF), jnp.float32),
        mesh=mesh,
        scratch_types=[
            pltpu.VMEM((NSUB, SUB), jnp.int32),
            pltpu.VMEM((NSUB, SUB), jnp.int32),
            pltpu.VMEM((B,), jnp.float32),
            pltpu.VMEM((B, DF), jnp.float32),
            pltpu.VMEM((ZROWS, DF), jnp.float32),
            pltpu.VMEM_SHARED((N_PAD, DF), jnp.float32),
            pltpu.SemaphoreType.DMA,
        ],
    )
    t_parts = agg(src, dst, x, vals)

    out = pl.pallas_call(
        _mm_body,
        out_shape=jax.ShapeDtypeStruct((N, DF), jnp.float32),
        grid=(N // BN,),
        in_specs=[
            pl.BlockSpec((KER, NC, BN, DF), lambda i: (0, 0, i, 0)),
            pl.BlockSpec((KER, DF, DF), lambda i: (0, 0, 0)),
            pl.BlockSpec((1, DF), lambda i: (0, 0)),
        ],
        out_specs=pl.BlockSpec((BN, DF), lambda i: (i, 0)),
    )(t_parts, wt, b2)
    return out


# speedup vs baseline: 2.6526x; 1.0848x over previous
import functools

import jax
import jax.numpy as jnp
from jax import lax
from jax.experimental import pallas as pl
from jax.experimental.pallas import tpu as pltpu
from jax.experimental.pallas import tpu_sc as plsc

N = 10000
E = 320000
DF = 128
KER = 4
EMB = 16

NC = 2
NS = 16
NW = NC * NS
EPT = E // NW
BA = 2000
SUBA = 80
NSUBA = BA // SUBA
NBATCHA = EPT // BA
B = 80
NBATCH = EPT // B
N_PAD = 10240
ROWS_PER_TILE = N_PAD // NS
ZROWS = 64


def _gather_spec_body(srcp, dstp, spec, d0_out, d1_out,
                      sidx, didx, d0, d1, sem):
    cid = lax.axis_index("c")
    sid = lax.axis_index("s")
    wid = cid * NS + sid

    def batch_body(i, carry):
        base = wid * EPT + i * BA
        for j in range(NSUBA):
            pltpu.sync_copy(srcp.at[pl.ds(base + j * SUBA, SUBA)], sidx.at[j])
            pltpu.sync_copy(dstp.at[pl.ds(base + j * SUBA, SUBA)], didx.at[j])
        descs = []
        for j in range(NSUBA):
            descs.append(pltpu.async_copy(
                spec.at[sidx.at[j]], d0.at[pl.ds(j * SUBA, SUBA)], sem))
            descs.append(pltpu.async_copy(
                spec.at[didx.at[j]], d1.at[pl.ds(j * SUBA, SUBA)], sem))
        for d in descs:
            d.wait()
        pltpu.sync_copy(d0, d0_out.at[pl.ds(base, BA)])
        pltpu.sync_copy(d1, d1_out.at[pl.ds(base, BA)])
        return carry
    lax.fori_loop(0, NBATCHA, batch_body, 0)


VB = 2560


def _vals_body(d0_ref, d1_ref, mu_ref, sig_ref, o_ref):
    diff = d0_ref[...] - d1_ref[...]
    sq = jnp.sum(diff * diff, axis=1)
    p = jnp.dot(diff, mu_ref[...], preferred_element_type=jnp.float32)
    for k in range(KER):
        m2k = jnp.sum(mu_ref[:, k] * mu_ref[:, k])
        qq = p[:, k] - 0.5 * (sq + m2k)
        o_ref[k, :] = jnp.exp(sig_ref[0, k] * qq)


def _agg_body(srcp, dstp, x, vals, t_out,
              sidx, didx, vb, xr, zbuf, acc, sem):
    cid = lax.axis_index("c")
    sid = lax.axis_index("s")
    wid = cid * NS + sid

    def zb_body(i, c):
        for r in range(DF // 16):
            zbuf[i, pl.ds(16 * r, 16)] = jnp.zeros((16,), jnp.float32)
        return c
    lax.fori_loop(0, ZROWS, zb_body, 0)

    for k in range(KER):
        for jj in range(ROWS_PER_TILE // ZROWS):
            pltpu.sync_copy(zbuf, acc.at[pl.ds(sid * ROWS_PER_TILE + jj * ZROWS, ZROWS)])
        plsc.subcore_barrier()

        def batch_body(i, carry):
            base = wid * EPT + i * B
            pltpu.sync_copy(srcp.at[pl.ds(base, B)], sidx.at[0])
            pltpu.sync_copy(dstp.at[pl.ds(base, B)], didx.at[0])
            pltpu.sync_copy(vals.at[pl.ds(k * E + base, B)], vb)
            pltpu.async_copy(x.at[didx.at[0]], xr, sem).wait()

            def scale_body(g, c):
                vvec = vb[pl.ds(g * 16, 16)]
                for t in range(16):
                    ve = vvec[t]
                    e = g * 16 + t
                    for r in range(DF // 16):
                        xr[e, pl.ds(16 * r, 16)] = xr[e, pl.ds(16 * r, 16)] * ve
                return c
            lax.fori_loop(0, B // 16, scale_body, 0)

            pltpu.sync_copy(xr, acc.at[sidx.at[0]], add=True)
            return carry
        lax.fori_loop(0, NBATCH, batch_body, 0)
        plsc.subcore_barrier()
        pltpu.sync_copy(acc.at[pl.ds(sid * ROWS_PER_TILE, ROWS_PER_TILE)],
                        t_out.at[k, cid, pl.ds(sid * ROWS_PER_TILE, ROWS_PER_TILE)])
        plsc.subcore_barrier()


BN = 400


def _mm_body(t_ref, w_ref, b_ref, o_ref):
    acc = jnp.zeros((BN, DF), jnp.float32)
    for k in range(KER):
        acc = acc + jnp.dot(t_ref[k, 0] + t_ref[k, 1], w_ref[k],
                            preferred_element_type=jnp.float32)
    o_ref[...] = acc + b_ref[...]


def kernel(x, edge_idx, spec_domain, weight, bias, mu, sig):
    src = edge_idx[0]
    dst = edge_idx[1]
    wt = jnp.transpose(weight, (2, 0, 1))
    b2 = bias[None, :]

    mesh = plsc.VectorSubcoreMesh(core_axis_name="c", subcore_axis_name="s")

    gather_spec = pl.kernel(
        _gather_spec_body,
        out_type=(jax.ShapeDtypeStruct((E, EMB), jnp.float32),
                  jax.ShapeDtypeStruct((E, EMB), jnp.float32)),
        mesh=mesh,
        scratch_types=[
            pltpu.VMEM((NSUBA, SUBA), jnp.int32),
            pltpu.VMEM((NSUBA, SUBA), jnp.int32),
            pltpu.VMEM((BA, EMB), jnp.float32),
            pltpu.VMEM((BA, EMB), jnp.float32),
            pltpu.SemaphoreType.DMA,
        ],
        compiler_params=pltpu.CompilerParams(use_tc_tiling_on_sc=False),
    )
    d0f, d1f = gather_spec(src, dst, spec_domain)

    vals4 = pl.pallas_call(
        _vals_body,
        out_shape=jax.ShapeDtypeStruct((KER, E), jnp.float32),
        grid=(E // VB,),
        in_specs=[
            pl.BlockSpec((VB, EMB), lambda i: (i, 0)),
            pl.BlockSpec((VB, EMB), lambda i: (i, 0)),
            pl.BlockSpec((EMB, KER), lambda i: (0, 0)),
            pl.BlockSpec((1, KER), lambda i: (0, 0)),
        ],
        out_specs=pl.BlockSpec((KER, VB), lambda i: (0, i)),
    )(d0f, d1f, mu, sig)
    vals = vals4.reshape(-1)

    agg = pl.kernel(
        _agg_body,
        out_type=jax.ShapeDtypeStruct((KER, NC, N_PAD, DF), jnp.float32),
        mesh=mesh,
        scratch_types=[
            pltpu.VMEM((1, B), jnp.int32),
            pltpu.VMEM((1, B), jnp.int32),
            pltpu.VMEM((B,), jnp.float32),
            pltpu.VMEM((B, DF), jnp.float32),
            pltpu.VMEM((ZROWS, DF), jnp.float32),
            pltpu.VMEM_SHARED((N_PAD, DF), jnp.float32),
            pltpu.SemaphoreType.DMA,
        ],
    )
    t_parts = agg(src, dst, x, vals)

    out = pl.pallas_call(
        _mm_body,
        out_shape=jax.ShapeDtypeStruct((N, DF), jnp.float32),
        grid=(N // BN,),
        in_specs=[
            pl.BlockSpec((KER, NC, BN, DF), lambda i: (0, 0, i, 0)),
            pl.BlockSpec((KER, DF, DF), lambda i: (0, 0, 0)),
            pl.BlockSpec((1, DF), lambda i: (0, 0)),
        ],
        out_specs=pl.BlockSpec((BN, DF), lambda i: (i, 0)),
    )(t_parts, wt, b2)
    return out
